# baseline (speedup 1.0000x reference)
import functools

import jax
import jax.numpy as jnp
from jax import lax
from jax.experimental import pallas as pl
from jax.experimental.pallas import tpu as pltpu
from jax.experimental.pallas import tpu_sc as plsc

N_HEADS = 4
N_NODES = 10000
EMB_DIM = 128
N_BATCH = 512
N_NEAREST = 8
N_SENTINELS = 8

QB = 128
NQ = 2 * N_BATCH
NROWS = N_HEADS * NQ

NWORKERS = 32
RPW = NROWS // NWORKERS
WPH = NQ // RPW
CHUNK = 32
LANES = 16


def _sc_gather_qf(embeds_flat, field_flat, nodes):
    mesh = plsc.VectorSubcoreMesh(core_axis_name="c", subcore_axis_name="s")

    @functools.partial(
        pl.kernel,
        out_type=[
            jax.ShapeDtypeStruct((NROWS, EMB_DIM), jnp.float32),
            jax.ShapeDtypeStruct((NROWS, EMB_DIM), jnp.float32),
        ],
        mesh=mesh,
        compiler_params=pltpu.CompilerParams(needs_layout_passes=False),
        scratch_types=[
            pltpu.VMEM((RPW,), jnp.int32),
            pltpu.VMEM((RPW, EMB_DIM), jnp.float32),
            pltpu.VMEM((RPW, EMB_DIM), jnp.float32),
            pltpu.SemaphoreType.DMA,
        ],
    )
    def k(e_hbm, f_hbm, nodes_hbm, q_out, f_out, idx_v, qrows_v, frows_v, sem):
        wid = lax.axis_index("s") * 2 + lax.axis_index("c")
        h = wid // WPH
        base = wid * RPW
        boff = (wid % WPH) * RPW
        pltpu.sync_copy(nodes_hbm.at[pl.ds(boff, RPW)], idx_v)
        off = h * N_NODES
        for i in range(RPW // LANES):
            sl = pl.ds(i * LANES, LANES)
            idx_v[sl] = idx_v[sl] + off
        cq = pltpu.async_copy(e_hbm.at[idx_v], qrows_v, sem)
        cq.wait()
        pltpu.sync_copy(qrows_v, q_out.at[pl.ds(base, RPW)])
        cf = pltpu.async_copy(f_hbm.at[idx_v], frows_v, sem)
        cf.wait()
        pltpu.sync_copy(frows_v, f_out.at[pl.ds(base, RPW)])

    return k(embeds_flat, field_flat, nodes)


def _topk_body(q_ref, e_ref, idx_ref):
    q = q_ref[0]
    e = e_ref[0]
    qn = jnp.sum(q * q, axis=1, keepdims=True)
    kn = jnp.sum(e * e, axis=1)
    prod = lax.dot_general(q, e, (((1,), (1,)), ((), ())),
                           preferred_element_type=jnp.float32)
    d2 = jnp.maximum(qn + kn[None, :] - 2.0 * prod, 0.0)
    n = d2.shape[1]
    iota = lax.broadcasted_iota(jnp.int32, d2.shape, 1)
    cols = []
    for _ in range(N_NEAREST + 1):
        m = jnp.min(d2, axis=1, keepdims=True)
        am = jnp.min(jnp.where(d2 == m, iota, n), axis=1)
        cols.append(am)
        d2 = jnp.where(iota == am[:, None], jnp.inf, d2)
    idx_ref[0] = jnp.stack(cols, axis=1)


def _topk(qrows, embeds):
    grid = (N_HEADS, NQ // QB)
    return pl.pallas_call(
        _topk_body,
        grid=grid,
        in_specs=[
            pl.BlockSpec((1, QB, EMB_DIM), lambda h, b: (h, b, 0)),
            pl.BlockSpec((1, N_NODES, EMB_DIM), lambda h, b: (h, 0, 0)),
        ],
        out_specs=pl.BlockSpec((1, QB, N_NEAREST + 1), lambda h, b: (h, b, 0)),
        out_shape=jax.ShapeDtypeStruct((N_HEADS, NQ, N_NEAREST + 1), jnp.int32),
    )(qrows, embeds)


def _sc_dots(embeds_flat, qrows, frows, samp_flat):
    mesh = plsc.VectorSubcoreMesh(core_axis_name="c", subcore_axis_name="s")
    K = N_NEAREST
    NCH = RPW // CHUNK

    @functools.partial(
        pl.kernel,
        out_type=[
            jax.ShapeDtypeStruct((NROWS * K,), jnp.float32),
            jax.ShapeDtypeStruct((NROWS * K,), jnp.float32),
        ],
        mesh=mesh,
        compiler_params=pltpu.CompilerParams(needs_layout_passes=False),
        scratch_types=[
            pltpu.VMEM((CHUNK * K,), jnp.int32),
            pltpu.VMEM((CHUNK, EMB_DIM), jnp.float32),
            pltpu.VMEM((CHUNK, EMB_DIM), jnp.float32),
            pltpu.VMEM((CHUNK * K, EMB_DIM), jnp.float32),
            pltpu.VMEM((CHUNK * K,), jnp.float32),
            pltpu.VMEM((CHUNK * K,), jnp.float32),
            pltpu.SemaphoreType.DMA,
        ],
    )
    def k(e_hbm, q_hbm, f_hbm, samp_hbm, d2_out, lg_out,
          idx_v, q_v, f_v, s_v, d2_v, lg_v, sem):
        wid = lax.axis_index("s") * 2 + lax.axis_index("c")
        h = wid // WPH
        off = h * N_NODES
        lane_iota = lax.iota(jnp.int32, LANES)
        for c in range(NCH):
            qbase = wid * RPW + c * CHUNK
            pltpu.sync_copy(q_hbm.at[pl.ds(qbase, CHUNK)], q_v)
            pltpu.sync_copy(f_hbm.at[pl.ds(qbase, CHUNK)], f_v)
            pltpu.sync_copy(samp_hbm.at[pl.ds(qbase * K, CHUNK * K)], idx_v)
            for i in range((CHUNK * K) // LANES):
                sl = pl.ds(i * LANES, LANES)
                idx_v[sl] = idx_v[sl] + off
            pltpu.async_copy(e_hbm.at[idx_v], s_v, sem).wait()

            for g in range((CHUNK * K) // LANES):
                rvec = g * LANES + lane_iota
                ivec = lax.shift_right_logical(rvec, 3)

                def dimstep(d, carry):
                    d2acc, lgacc = carry
                    dsplat = jnp.zeros((LANES,), jnp.int32) + d
                    sval = plsc.load_gather(s_v, [rvec, dsplat])
                    qval = plsc.load_gather(q_v, [ivec, dsplat])
                    fval = plsc.load_gather(f_v, [ivec, dsplat])
                    dv = qval - sval
                    return d2acc + dv * dv, lgacc + dv * fval

                z = jnp.zeros((LANES,), jnp.float32)
                d2acc, lgacc = lax.fori_loop(0, EMB_DIM, dimstep, (z, z))
                gsl = pl.ds(g * LANES, LANES)
                d2_v[gsl] = d2acc
                lg_v[gsl] = lgacc
            pltpu.sync_copy(d2_v, d2_out.at[pl.ds(qbase * K, CHUNK * K)])
            pltpu.sync_copy(lg_v, lg_out.at[pl.ds(qbase * K, CHUNK * K)])

    return k(embeds_flat, qrows, frows, samp_flat)


def _combine_body(d2_ref, lg_ref, out_ref):
    d2 = d2_ref[...]
    lg = lg_ref[...]
    dist = jnp.sqrt(d2)
    e = jnp.exp(1.0 - dist)
    num = jnp.sum(e * lg, axis=2)
    den = jnp.sum(e, axis=2)
    num_t = num[:, :N_BATCH] + num[:, N_BATCH:]
    den_t = den[:, :N_BATCH] + den[:, N_BATCH:] + float(N_SENTINELS)
    softmin = num_t / den_t
    preds = jnp.mean(softmin, axis=0)
    out_ref[...] = (1.0 / (1.0 + jnp.exp(-preds)))[None, :]


def _combine(d2, lg):
    return pl.pallas_call(
        _combine_body,
        out_shape=jax.ShapeDtypeStruct((1, N_BATCH), jnp.float32),
    )(d2, lg)


@jax.jit
def kernel(adj_t, edges, embeds, field):
    nodes = jnp.concatenate([edges[0], edges[1]]).astype(jnp.int32)
    embeds_flat = embeds.reshape(N_HEADS * N_NODES, EMB_DIM)
    field_flat = field.reshape(N_HEADS * N_NODES, EMB_DIM)

    qrows, frows = _sc_gather_qf(embeds_flat, field_flat, nodes)

    idx9 = _topk(qrows.reshape(N_HEADS, NQ, EMB_DIM), embeds)
    samples = idx9[:, :, 1:]
    samp_flat = samples.reshape(NROWS * N_NEAREST)

    d2, lg = _sc_dots(embeds_flat, qrows, frows, samp_flat)
    out = _combine(d2.reshape(N_HEADS, NQ, N_NEAREST),
                   lg.reshape(N_HEADS, NQ, N_NEAREST))
    return out.reshape(N_BATCH)


# speedup vs baseline: 1.0303x; 1.0303x over previous
import functools

import jax
import jax.numpy as jnp
from jax import lax
from jax.experimental import pallas as pl
from jax.experimental.pallas import tpu as pltpu
from jax.experimental.pallas import tpu_sc as plsc

N_HEADS = 4
N_NODES = 10000
EMB_DIM = 128
N_BATCH = 512
N_NEAREST = 8
N_SENTINELS = 8

QB = 128
NQ = 2 * N_BATCH
NROWS = N_HEADS * NQ

NWORKERS = 32
RPW = NROWS // NWORKERS
WPH = NQ // RPW
CHUNK = 32
LANES = 16


def _sc_gather_qf(embeds_flat, field_flat, nodes):
    mesh = plsc.VectorSubcoreMesh(core_axis_name="c", subcore_axis_name="s")

    @functools.partial(
        pl.kernel,
        out_type=[
            jax.ShapeDtypeStruct((NROWS, EMB_DIM), jnp.float32),
            jax.ShapeDtypeStruct((NROWS, EMB_DIM), jnp.float32),
        ],
        mesh=mesh,
        compiler_params=pltpu.CompilerParams(needs_layout_passes=False),
        scratch_types=[
            pltpu.VMEM((RPW,), jnp.int32),
            pltpu.VMEM((RPW, EMB_DIM), jnp.float32),
            pltpu.VMEM((RPW, EMB_DIM), jnp.float32),
            pltpu.SemaphoreType.DMA,
        ],
    )
    def k(e_hbm, f_hbm, nodes_hbm, q_out, f_out, idx_v, qrows_v, frows_v, sem):
        wid = lax.axis_index("s") * 2 + lax.axis_index("c")
        h = wid // WPH
        base = wid * RPW
        boff = (wid % WPH) * RPW
        pltpu.sync_copy(nodes_hbm.at[pl.ds(boff, RPW)], idx_v)
        off = h * N_NODES
        for i in range(RPW // LANES):
            sl = pl.ds(i * LANES, LANES)
            idx_v[sl] = idx_v[sl] + off
        cq = pltpu.async_copy(e_hbm.at[idx_v], qrows_v, sem)
        cq.wait()
        pltpu.sync_copy(qrows_v, q_out.at[pl.ds(base, RPW)])
        cf = pltpu.async_copy(f_hbm.at[idx_v], frows_v, sem)
        cf.wait()
        pltpu.sync_copy(frows_v, f_out.at[pl.ds(base, RPW)])

    return k(embeds_flat, field_flat, nodes)


def _topk_body(q_ref, e_ref, idx_ref):
    q = q_ref[0]
    e = e_ref[0]
    qn = jnp.sum(q * q, axis=1, keepdims=True)
    kn = jnp.sum(e * e, axis=1)
    prod = lax.dot_general(q, e, (((1,), (1,)), ((), ())),
                           preferred_element_type=jnp.float32)
    d2 = jnp.maximum(qn + kn[None, :] - 2.0 * prod, 0.0)
    n = d2.shape[1]
    iota = lax.broadcasted_iota(jnp.int32, d2.shape, 1)
    cols = []
    for _ in range(N_NEAREST + 1):
        m = jnp.min(d2, axis=1, keepdims=True)
        am = jnp.min(jnp.where(d2 == m, iota, n), axis=1)
        cols.append(am)
        d2 = jnp.where(iota == am[:, None], jnp.inf, d2)
    idx_ref[0] = jnp.stack(cols, axis=1)


def _topk(qrows, embeds):
    grid = (N_HEADS, NQ // QB)
    return pl.pallas_call(
        _topk_body,
        grid=grid,
        in_specs=[
            pl.BlockSpec((1, QB, EMB_DIM), lambda h, b: (h, b, 0)),
            pl.BlockSpec((1, N_NODES, EMB_DIM), lambda h, b: (h, 0, 0)),
        ],
        out_specs=pl.BlockSpec((1, QB, N_NEAREST + 1), lambda h, b: (h, b, 0)),
        out_shape=jax.ShapeDtypeStruct((N_HEADS, NQ, N_NEAREST + 1), jnp.int32),
    )(qrows, embeds)


def _sc_dots(embeds_flat, qrows, frows, samp_flat):
    mesh = plsc.VectorSubcoreMesh(core_axis_name="c", subcore_axis_name="s")
    K = N_NEAREST
    NCH = RPW // CHUNK

    @functools.partial(
        pl.kernel,
        out_type=[
            jax.ShapeDtypeStruct((NROWS * K,), jnp.float32),
            jax.ShapeDtypeStruct((NROWS * K,), jnp.float32),
        ],
        mesh=mesh,
        compiler_params=pltpu.CompilerParams(needs_layout_passes=False),
        scratch_types=[
            pltpu.VMEM((CHUNK * K,), jnp.int32),
            pltpu.VMEM((CHUNK, EMB_DIM), jnp.float32),
            pltpu.VMEM((CHUNK, EMB_DIM), jnp.float32),
            pltpu.VMEM((CHUNK * K, EMB_DIM), jnp.float32),
            pltpu.VMEM((CHUNK * K,), jnp.float32),
            pltpu.VMEM((CHUNK * K,), jnp.float32),
            pltpu.SemaphoreType.DMA,
        ],
    )
    def k(e_hbm, q_hbm, f_hbm, samp_hbm, d2_out, lg_out,
          idx_v, q_v, f_v, s_v, d2_v, lg_v, sem):
        wid = lax.axis_index("s") * 2 + lax.axis_index("c")
        h = wid // WPH
        off = h * N_NODES
        lane_iota = lax.iota(jnp.int32, LANES)

        def chunk_body(c, carry):
            qbase = wid * RPW + c * CHUNK
            pltpu.sync_copy(q_hbm.at[pl.ds(qbase, CHUNK)], q_v)
            pltpu.sync_copy(f_hbm.at[pl.ds(qbase, CHUNK)], f_v)
            pltpu.sync_copy(samp_hbm.at[pl.ds(qbase * K, CHUNK * K)], idx_v)
            for i in range((CHUNK * K) // LANES):
                sl = pl.ds(i * LANES, LANES)
                idx_v[sl] = idx_v[sl] + off
            pltpu.async_copy(e_hbm.at[idx_v], s_v, sem).wait()

            for g in range((CHUNK * K) // LANES):
                rvec = g * LANES + lane_iota
                ivec = lax.shift_right_logical(rvec, 3)

                def dimstep(dd, carry):
                    d2acc, lgacc = carry
                    dbase = jnp.zeros((LANES,), jnp.int32) + dd * LANES
                    for u in range(LANES):
                        dsplat = dbase + u
                        sval = plsc.load_gather(s_v, [rvec, dsplat])
                        qval = plsc.load_gather(q_v, [ivec, dsplat])
                        fval = plsc.load_gather(f_v, [ivec, dsplat])
                        dv = qval - sval
                        d2acc = d2acc + dv * dv
                        lgacc = lgacc + dv * fval
                    return d2acc, lgacc

                z = jnp.zeros((LANES,), jnp.float32)
                d2acc, lgacc = lax.fori_loop(0, EMB_DIM // LANES, dimstep,
                                             (z, z))
                gsl = pl.ds(g * LANES, LANES)
                d2_v[gsl] = d2acc
                lg_v[gsl] = lgacc
            pltpu.sync_copy(d2_v, d2_out.at[pl.ds(qbase * K, CHUNK * K)])
            pltpu.sync_copy(lg_v, lg_out.at[pl.ds(qbase * K, CHUNK * K)])
            return carry

        lax.fori_loop(0, NCH, chunk_body, 0)

    return k(embeds_flat, qrows, frows, samp_flat)


def _combine_body(d2_ref, lg_ref, out_ref):
    d2 = d2_ref[...]
    lg = lg_ref[...]
    dist = jnp.sqrt(d2)
    e = jnp.exp(1.0 - dist)
    num = jnp.sum(e * lg, axis=2)
    den = jnp.sum(e, axis=2)
    num_t = num[:, :N_BATCH] + num[:, N_BATCH:]
    den_t = den[:, :N_BATCH] + den[:, N_BATCH:] + float(N_SENTINELS)
    softmin = num_t / den_t
    preds = jnp.mean(softmin, axis=0)
    out_ref[...] = (1.0 / (1.0 + jnp.exp(-preds)))[None, :]


def _combine(d2, lg):
    return pl.pallas_call(
        _combine_body,
        out_shape=jax.ShapeDtypeStruct((1, N_BATCH), jnp.float32),
    )(d2, lg)


@jax.jit
def kernel(adj_t, edges, embeds, field):
    nodes = jnp.concatenate([edges[0], edges[1]]).astype(jnp.int32)
    embeds_flat = embeds.reshape(N_HEADS * N_NODES, EMB_DIM)
    field_flat = field.reshape(N_HEADS * N_NODES, EMB_DIM)

    qrows, frows = _sc_gather_qf(embeds_flat, field_flat, nodes)

    idx9 = _topk(qrows.reshape(N_HEADS, NQ, EMB_DIM), embeds)
    samples = idx9[:, :, 1:]
    samp_flat = samples.reshape(NROWS * N_NEAREST)

    d2, lg = _sc_dots(embeds_flat, qrows, frows, samp_flat)
    out = _combine(d2.reshape(N_HEADS, NQ, N_NEAREST),
                   lg.reshape(N_HEADS, NQ, N_NEAREST))
    return out.reshape(N_BATCH)


# speedup vs baseline: 1.3779x; 1.3373x over previous
import functools

import jax
import jax.numpy as jnp
from jax import lax
from jax.experimental import pallas as pl
from jax.experimental.pallas import tpu as pltpu
from jax.experimental.pallas import tpu_sc as plsc

N_HEADS = 4
N_NODES = 10000
EMB_DIM = 128
N_BATCH = 512
N_NEAREST = 8
N_SENTINELS = 8

QB = 128
NQ = 2 * N_BATCH
NROWS = N_HEADS * NQ

NWORKERS = 32
RPW = NROWS // NWORKERS
WPH = NQ // RPW
CHUNK = 32
LANES = 16


def _sc_gather_qf(embeds_flat, field_flat, nodes):
    mesh = plsc.VectorSubcoreMesh(core_axis_name="c", subcore_axis_name="s")

    @functools.partial(
        pl.kernel,
        out_type=[
            jax.ShapeDtypeStruct((NROWS, EMB_DIM), jnp.float32),
            jax.ShapeDtypeStruct((NROWS, EMB_DIM), jnp.float32),
        ],
        mesh=mesh,
        compiler_params=pltpu.CompilerParams(needs_layout_passes=False),
        scratch_types=[
            pltpu.VMEM((RPW,), jnp.int32),
            pltpu.VMEM((RPW, EMB_DIM), jnp.float32),
            pltpu.VMEM((RPW, EMB_DIM), jnp.float32),
            pltpu.SemaphoreType.DMA,
        ],
    )
    def k(e_hbm, f_hbm, nodes_hbm, q_out, f_out, idx_v, qrows_v, frows_v, sem):
        wid = lax.axis_index("s") * 2 + lax.axis_index("c")
        h = wid // WPH
        base = wid * RPW
        boff = (wid % WPH) * RPW
        pltpu.sync_copy(nodes_hbm.at[pl.ds(boff, RPW)], idx_v)
        off = h * N_NODES
        for i in range(RPW // LANES):
            sl = pl.ds(i * LANES, LANES)
            idx_v[sl] = idx_v[sl] + off
        cq = pltpu.async_copy(e_hbm.at[idx_v], qrows_v, sem)
        cq.wait()
        pltpu.sync_copy(qrows_v, q_out.at[pl.ds(base, RPW)])
        cf = pltpu.async_copy(f_hbm.at[idx_v], frows_v, sem)
        cf.wait()
        pltpu.sync_copy(frows_v, f_out.at[pl.ds(base, RPW)])

    return k(embeds_flat, field_flat, nodes)


def _topk_body(q_ref, e_ref, idx_ref):
    q = q_ref[0]
    e = e_ref[0]
    qn = jnp.sum(q * q, axis=1, keepdims=True)
    kn = jnp.sum(e * e, axis=1)
    prod = lax.dot_general(q, e, (((1,), (1,)), ((), ())),
                           preferred_element_type=jnp.float32)
    d2 = jnp.maximum(qn + kn[None, :] - 2.0 * prod, 0.0)
    n = d2.shape[1]
    iota = lax.broadcasted_iota(jnp.int32, d2.shape, 1)
    cols = []
    for _ in range(N_NEAREST + 1):
        m = jnp.min(d2, axis=1, keepdims=True)
        am = jnp.min(jnp.where(d2 == m, iota, n), axis=1)
        cols.append(am)
        d2 = jnp.where(iota == am[:, None], jnp.inf, d2)
    idx_ref[0] = jnp.stack(cols, axis=1)


def _topk(qrows, embeds):
    grid = (N_HEADS, NQ // QB)
    return pl.pallas_call(
        _topk_body,
        grid=grid,
        in_specs=[
            pl.BlockSpec((1, QB, EMB_DIM), lambda h, b: (h, b, 0)),
            pl.BlockSpec((1, N_NODES, EMB_DIM), lambda h, b: (h, 0, 0)),
        ],
        out_specs=pl.BlockSpec((1, QB, N_NEAREST + 1), lambda h, b: (h, b, 0)),
        out_shape=jax.ShapeDtypeStruct((N_HEADS, NQ, N_NEAREST + 1), jnp.int32),
    )(qrows, embeds)


def _sc_dots(embeds_flat, qrows, frows, samp_flat):
    mesh = plsc.VectorSubcoreMesh(core_axis_name="c", subcore_axis_name="s")
    K = N_NEAREST
    NCH = RPW // CHUNK

    @functools.partial(
        pl.kernel,
        out_type=[
            jax.ShapeDtypeStruct((NROWS * K,), jnp.float32),
            jax.ShapeDtypeStruct((NROWS * K,), jnp.float32),
        ],
        mesh=mesh,
        compiler_params=pltpu.CompilerParams(needs_layout_passes=False),
        scratch_types=[
            pltpu.VMEM((CHUNK * K,), jnp.int32),
            pltpu.VMEM((CHUNK, EMB_DIM), jnp.float32),
            pltpu.VMEM((CHUNK, EMB_DIM), jnp.float32),
            pltpu.VMEM((CHUNK * K, EMB_DIM), jnp.float32),
            pltpu.VMEM((CHUNK * K,), jnp.float32),
            pltpu.VMEM((CHUNK * K,), jnp.float32),
            pltpu.SemaphoreType.DMA,
        ],
    )
    def k(e_hbm, q_hbm, f_hbm, samp_hbm, d2_out, lg_out,
          idx_v, q_v, f_v, s_v, d2_v, lg_v, sem):
        wid = lax.axis_index("s") * 2 + lax.axis_index("c")
        h = wid // WPH
        off = h * N_NODES
        lane_iota = lax.iota(jnp.int32, LANES)

        def chunk_body(c, carry):
            qbase = wid * RPW + c * CHUNK
            pltpu.sync_copy(q_hbm.at[pl.ds(qbase, CHUNK)], q_v)
            pltpu.sync_copy(f_hbm.at[pl.ds(qbase, CHUNK)], f_v)
            pltpu.sync_copy(samp_hbm.at[pl.ds(qbase * K, CHUNK * K)], idx_v)
            for i in range((CHUNK * K) // LANES):
                sl = pl.ds(i * LANES, LANES)
                idx_v[sl] = idx_v[sl] + off
            pltpu.async_copy(e_hbm.at[idx_v], s_v, sem).wait()

            def pair_body(p, carry):
                resd2 = jnp.zeros((LANES,), jnp.float32)
                reslg = jnp.zeros((LANES,), jnp.float32)
                for qq in range(2):
                    i = 2 * p + qq
                    qregs = [q_v[i, pl.ds(j * LANES, LANES)]
                             for j in range(EMB_DIM // LANES)]
                    fregs = [f_v[i, pl.ds(j * LANES, LANES)]
                             for j in range(EMB_DIM // LANES)]
                    for kk in range(K):
                        t = i * K + kk
                        d2acc = jnp.zeros((LANES,), jnp.float32)
                        lgacc = jnp.zeros((LANES,), jnp.float32)
                        for j in range(EMB_DIM // LANES):
                            sv = s_v[t, pl.ds(j * LANES, LANES)]
                            dv = qregs[j] - sv
                            d2acc = d2acc + dv * dv
                            lgacc = lgacc + dv * fregs[j]
                        hit = lane_iota == (qq * K + kk)
                        resd2 = jnp.where(hit, jnp.sum(d2acc), resd2)
                        reslg = jnp.where(hit, jnp.sum(lgacc), reslg)
                gsl = pl.ds(p * LANES, LANES)
                d2_v[gsl] = resd2
                lg_v[gsl] = reslg
                return carry

            lax.fori_loop(0, (CHUNK * K) // LANES, pair_body, 0)
            pltpu.sync_copy(d2_v, d2_out.at[pl.ds(qbase * K, CHUNK * K)])
            pltpu.sync_copy(lg_v, lg_out.at[pl.ds(qbase * K, CHUNK * K)])
            return carry

        lax.fori_loop(0, NCH, chunk_body, 0)

    return k(embeds_flat, qrows, frows, samp_flat)


def _combine_body(d2_ref, lg_ref, out_ref):
    d2 = d2_ref[...]
    lg = lg_ref[...]
    dist = jnp.sqrt(d2)
    e = jnp.exp(1.0 - dist)
    num = jnp.sum(e * lg, axis=2)
    den = jnp.sum(e, axis=2)
    num_t = num[:, :N_BATCH] + num[:, N_BATCH:]
    den_t = den[:, :N_BATCH] + den[:, N_BATCH:] + float(N_SENTINELS)
    softmin = num_t / den_t
    preds = jnp.mean(softmin, axis=0)
    out_ref[...] = (1.0 / (1.0 + jnp.exp(-preds)))[None, :]


def _combine(d2, lg):
    return pl.pallas_call(
        _combine_body,
        out_shape=jax.ShapeDtypeStruct((1, N_BATCH), jnp.float32),
    )(d2, lg)


@jax.jit
def kernel(adj_t, edges, embeds, field):
    nodes = jnp.concatenate([edges[0], edges[1]]).astype(jnp.int32)
    embeds_flat = embeds.reshape(N_HEADS * N_NODES, EMB_DIM)
    field_flat = field.reshape(N_HEADS * N_NODES, EMB_DIM)

    qrows, frows = _sc_gather_qf(embeds_flat, field_flat, nodes)

    idx9 = _topk(qrows.reshape(N_HEADS, NQ, EMB_DIM), embeds)
    samples = idx9[:, :, 1:]
    samp_flat = samples.reshape(NROWS * N_NEAREST)

    d2, lg = _sc_dots(embeds_flat, qrows, frows, samp_flat)
    out = _combine(d2.reshape(N_HEADS, NQ, N_NEAREST),
                   lg.reshape(N_HEADS, NQ, N_NEAREST))
    return out.reshape(N_BATCH)


# speedup vs baseline: 1.5761x; 1.1439x over previous
import functools

import jax
import jax.numpy as jnp
from jax import lax
from jax.experimental import pallas as pl
from jax.experimental.pallas import tpu as pltpu
from jax.experimental.pallas import tpu_sc as plsc

N_HEADS = 4
N_NODES = 10000
EMB_DIM = 128
N_BATCH = 512
N_NEAREST = 8
N_SENTINELS = 8

QB = 128
NQ = 2 * N_BATCH
NROWS = N_HEADS * NQ

NWORKERS = 32
RPW = NROWS // NWORKERS
WPH = NQ // RPW
CHUNK = 32
LANES = 16


def _sc_gather_qf(embeds_flat, field_flat, nodes):
    mesh = plsc.VectorSubcoreMesh(core_axis_name="c", subcore_axis_name="s")

    @functools.partial(
        pl.kernel,
        out_type=[
            jax.ShapeDtypeStruct((NROWS, EMB_DIM), jnp.float32),
            jax.ShapeDtypeStruct((NROWS, EMB_DIM), jnp.float32),
        ],
        mesh=mesh,
        compiler_params=pltpu.CompilerParams(needs_layout_passes=False),
        scratch_types=[
            pltpu.VMEM((RPW,), jnp.int32),
            pltpu.VMEM((RPW, EMB_DIM), jnp.float32),
            pltpu.VMEM((RPW, EMB_DIM), jnp.float32),
            pltpu.SemaphoreType.DMA,
        ],
    )
    def k(e_hbm, f_hbm, nodes_hbm, q_out, f_out, idx_v, qrows_v, frows_v, sem):
        wid = lax.axis_index("s") * 2 + lax.axis_index("c")
        h = wid // WPH
        base = wid * RPW
        boff = (wid % WPH) * RPW
        pltpu.sync_copy(nodes_hbm.at[pl.ds(boff, RPW)], idx_v)
        off = h * N_NODES
        for i in range(RPW // LANES):
            sl = pl.ds(i * LANES, LANES)
            idx_v[sl] = idx_v[sl] + off
        cq = pltpu.async_copy(e_hbm.at[idx_v], qrows_v, sem)
        cq.wait()
        pltpu.sync_copy(qrows_v, q_out.at[pl.ds(base, RPW)])
        cf = pltpu.async_copy(f_hbm.at[idx_v], frows_v, sem)
        cf.wait()
        pltpu.sync_copy(frows_v, f_out.at[pl.ds(base, RPW)])

    return k(embeds_flat, field_flat, nodes)


def _topk_body(q_ref, e_ref, nodes_ref, idx_ref):
    q = q_ref[0]
    e = e_ref[0]
    qn = jnp.sum(q * q, axis=1, keepdims=True)
    kn = jnp.sum(e * e, axis=1)
    prod = lax.dot_general(q, e, (((1,), (1,)), ((), ())),
                           preferred_element_type=jnp.float32)
    d2 = jnp.maximum(qn + kn[None, :] - 2.0 * prod, 0.0)
    iota = lax.broadcasted_iota(jnp.int32, d2.shape, 1)
    nodeb = nodes_ref[0, 0]
    d2 = jnp.where(iota == nodeb[:, None], jnp.inf, d2)
    cols = []
    for _ in range(N_NEAREST):
        am = jnp.argmin(d2, axis=1).astype(jnp.int32)
        cols.append(am)
        d2 = jnp.where(iota == am[:, None], jnp.inf, d2)
    idx_ref[0] = jnp.stack(cols, axis=1)


def _topk(qrows, embeds, nodes3):
    grid = (N_HEADS, NQ // QB)
    return pl.pallas_call(
        _topk_body,
        grid=grid,
        in_specs=[
            pl.BlockSpec((1, QB, EMB_DIM), lambda h, b: (h, b, 0)),
            pl.BlockSpec((1, N_NODES, EMB_DIM), lambda h, b: (h, 0, 0)),
            pl.BlockSpec((1, 1, QB), lambda h, b: (b, 0, 0)),
        ],
        out_specs=pl.BlockSpec((1, QB, N_NEAREST), lambda h, b: (h, b, 0)),
        out_shape=jax.ShapeDtypeStruct((N_HEADS, NQ, N_NEAREST), jnp.int32),
    )(qrows, embeds, nodes3)


def _sc_dots(embeds_flat, qrows, frows, samp_flat):
    mesh = plsc.VectorSubcoreMesh(core_axis_name="c", subcore_axis_name="s")
    K = N_NEAREST
    NCH = RPW // CHUNK

    @functools.partial(
        pl.kernel,
        out_type=[
            jax.ShapeDtypeStruct((NROWS * K,), jnp.float32),
            jax.ShapeDtypeStruct((NROWS * K,), jnp.float32),
        ],
        mesh=mesh,
        compiler_params=pltpu.CompilerParams(needs_layout_passes=False),
        scratch_types=[
            pltpu.VMEM((CHUNK * K,), jnp.int32),
            pltpu.VMEM((CHUNK, EMB_DIM), jnp.float32),
            pltpu.VMEM((CHUNK, EMB_DIM), jnp.float32),
            pltpu.VMEM((CHUNK * K, EMB_DIM), jnp.float32),
            pltpu.VMEM((CHUNK * K,), jnp.float32),
            pltpu.VMEM((CHUNK * K,), jnp.float32),
            pltpu.SemaphoreType.DMA,
        ],
    )
    def k(e_hbm, q_hbm, f_hbm, samp_hbm, d2_out, lg_out,
          idx_v, q_v, f_v, s_v, d2_v, lg_v, sem):
        wid = lax.axis_index("s") * 2 + lax.axis_index("c")
        h = wid // WPH
        off = h * N_NODES
        lane_iota = lax.iota(jnp.int32, LANES)

        def chunk_body(c, carry):
            qbase = wid * RPW + c * CHUNK
            pltpu.sync_copy(q_hbm.at[pl.ds(qbase, CHUNK)], q_v)
            pltpu.sync_copy(f_hbm.at[pl.ds(qbase, CHUNK)], f_v)
            pltpu.sync_copy(samp_hbm.at[pl.ds(qbase * K, CHUNK * K)], idx_v)
            for i in range((CHUNK * K) // LANES):
                sl = pl.ds(i * LANES, LANES)
                idx_v[sl] = idx_v[sl] + off
            pltpu.async_copy(e_hbm.at[idx_v], s_v, sem).wait()

            def pair_body(p, carry):
                resd2 = jnp.zeros((LANES,), jnp.float32)
                reslg = jnp.zeros((LANES,), jnp.float32)
                for qq in range(2):
                    i = 2 * p + qq
                    qregs = [q_v[i, pl.ds(j * LANES, LANES)]
                             for j in range(EMB_DIM // LANES)]
                    fregs = [f_v[i, pl.ds(j * LANES, LANES)]
                             for j in range(EMB_DIM // LANES)]
                    for kk in range(K):
                        t = i * K + kk
                        d2acc = jnp.zeros((LANES,), jnp.float32)
                        lgacc = jnp.zeros((LANES,), jnp.float32)
                        for j in range(EMB_DIM // LANES):
                            sv = s_v[t, pl.ds(j * LANES, LANES)]
                            dv = qregs[j] - sv
                            d2acc = d2acc + dv * dv
                            lgacc = lgacc + dv * fregs[j]
                        hit = lane_iota == (qq * K + kk)
                        resd2 = jnp.where(hit, jnp.sum(d2acc), resd2)
                        reslg = jnp.where(hit, jnp.sum(lgacc), reslg)
                gsl = pl.ds(p * LANES, LANES)
                d2_v[gsl] = resd2
                lg_v[gsl] = reslg
                return carry

            lax.fori_loop(0, (CHUNK * K) // LANES, pair_body, 0)
            pltpu.sync_copy(d2_v, d2_out.at[pl.ds(qbase * K, CHUNK * K)])
            pltpu.sync_copy(lg_v, lg_out.at[pl.ds(qbase * K, CHUNK * K)])
            return carry

        lax.fori_loop(0, NCH, chunk_body, 0)

    return k(embeds_flat, qrows, frows, samp_flat)


def _combine_body(d2_ref, lg_ref, out_ref):
    d2 = d2_ref[...]
    lg = lg_ref[...]
    dist = jnp.sqrt(d2)
    e = jnp.exp(1.0 - dist)
    num = jnp.sum(e * lg, axis=2)
    den = jnp.sum(e, axis=2)
    num_t = num[:, :N_BATCH] + num[:, N_BATCH:]
    den_t = den[:, :N_BATCH] + den[:, N_BATCH:] + float(N_SENTINELS)
    softmin = num_t / den_t
    preds = jnp.mean(softmin, axis=0)
    out_ref[...] = (1.0 / (1.0 + jnp.exp(-preds)))[None, :]


def _combine(d2, lg):
    return pl.pallas_call(
        _combine_body,
        out_shape=jax.ShapeDtypeStruct((1, N_BATCH), jnp.float32),
    )(d2, lg)


@jax.jit
def kernel(adj_t, edges, embeds, field):
    nodes = jnp.concatenate([edges[0], edges[1]]).astype(jnp.int32)
    embeds_flat = embeds.reshape(N_HEADS * N_NODES, EMB_DIM)
    field_flat = field.reshape(N_HEADS * N_NODES, EMB_DIM)

    qrows, frows = _sc_gather_qf(embeds_flat, field_flat, nodes)

    nodes3 = nodes.reshape(NQ // QB, 1, QB)
    samples = _topk(qrows.reshape(N_HEADS, NQ, EMB_DIM), embeds, nodes3)
    samp_flat = samples.reshape(NROWS * N_NEAREST)

    d2, lg = _sc_dots(embeds_flat, qrows, frows, samp_flat)
    out = _combine(d2.reshape(N_HEADS, NQ, N_NEAREST),
                   lg.reshape(N_HEADS, NQ, N_NEAREST))
    return out.reshape(N_BATCH)


# speedup vs baseline: 1.6797x; 1.0657x over previous
import functools

import jax
import jax.numpy as jnp
from jax import lax
from jax.experimental import pallas as pl
from jax.experimental.pallas import tpu as pltpu
from jax.experimental.pallas import tpu_sc as plsc

N_HEADS = 4
N_NODES = 10000
EMB_DIM = 128
N_BATCH = 512
N_NEAREST = 8
N_SENTINELS = 8

QB = 256
NQ = 2 * N_BATCH
NROWS = N_HEADS * NQ

NWORKERS = 32
RPW = NROWS // NWORKERS
WPH = NQ // RPW
CHUNK = 32
LANES = 16


def _sc_gather_qf(embeds_flat, field_flat, nodes):
    mesh = plsc.VectorSubcoreMesh(core_axis_name="c", subcore_axis_name="s")

    @functools.partial(
        pl.kernel,
        out_type=[
            jax.ShapeDtypeStruct((NROWS, EMB_DIM), jnp.float32),
            jax.ShapeDtypeStruct((NROWS, EMB_DIM), jnp.float32),
        ],
        mesh=mesh,
        compiler_params=pltpu.CompilerParams(needs_layout_passes=False),
        scratch_types=[
            pltpu.VMEM((RPW,), jnp.int32),
            pltpu.VMEM((RPW, EMB_DIM), jnp.float32),
            pltpu.VMEM((RPW, EMB_DIM), jnp.float32),
            pltpu.SemaphoreType.DMA,
        ],
    )
    def k(e_hbm, f_hbm, nodes_hbm, q_out, f_out, idx_v, qrows_v, frows_v, sem):
        wid = lax.axis_index("s") * 2 + lax.axis_index("c")
        h = wid // WPH
        base = wid * RPW
        boff = (wid % WPH) * RPW
        pltpu.sync_copy(nodes_hbm.at[pl.ds(boff, RPW)], idx_v)
        off = h * N_NODES
        for i in range(RPW // LANES):
            sl = pl.ds(i * LANES, LANES)
            idx_v[sl] = idx_v[sl] + off
        cq = pltpu.async_copy(e_hbm.at[idx_v], qrows_v, sem)
        cq.wait()
        pltpu.sync_copy(qrows_v, q_out.at[pl.ds(base, RPW)])
        cf = pltpu.async_copy(f_hbm.at[idx_v], frows_v, sem)
        cf.wait()
        pltpu.sync_copy(frows_v, f_out.at[pl.ds(base, RPW)])

    return k(embeds_flat, field_flat, nodes)


def _topk_body(q_ref, e_ref, nodes_ref, idx_ref):
    q = q_ref[0]
    e = e_ref[0]
    qn = jnp.sum(q * q, axis=1, keepdims=True)
    kn = jnp.sum(e * e, axis=1)
    prod = lax.dot_general(q, e, (((1,), (1,)), ((), ())),
                           preferred_element_type=jnp.float32)
    d2 = jnp.maximum(qn + kn[None, :] - 2.0 * prod, 0.0)
    iota = lax.broadcasted_iota(jnp.int32, d2.shape, 1)
    nodeb = nodes_ref[0, 0]
    d2 = jnp.where(iota == nodeb[:, None], jnp.inf, d2)
    cols = []
    for _ in range(N_NEAREST):
        am = jnp.argmin(d2, axis=1).astype(jnp.int32)
        cols.append(am)
        d2 = jnp.where(iota == am[:, None], jnp.inf, d2)
    idx_ref[0] = jnp.stack(cols, axis=1)


def _topk(qrows, embeds, nodes3):
    grid = (N_HEADS, NQ // QB)
    return pl.pallas_call(
        _topk_body,
        grid=grid,
        in_specs=[
            pl.BlockSpec((1, QB, EMB_DIM), lambda h, b: (h, b, 0)),
            pl.BlockSpec((1, N_NODES, EMB_DIM), lambda h, b: (h, 0, 0)),
            pl.BlockSpec((1, 1, QB), lambda h, b: (b, 0, 0)),
        ],
        out_specs=pl.BlockSpec((1, QB, N_NEAREST), lambda h, b: (h, b, 0)),
        out_shape=jax.ShapeDtypeStruct((N_HEADS, NQ, N_NEAREST), jnp.int32),
    )(qrows, embeds, nodes3)


def _sc_dots(embeds_flat, qrows, frows, samp_flat):
    mesh = plsc.VectorSubcoreMesh(core_axis_name="c", subcore_axis_name="s")
    K = N_NEAREST
    NCH = RPW // CHUNK

    @functools.partial(
        pl.kernel,
        out_type=[
            jax.ShapeDtypeStruct((NROWS * K,), jnp.float32),
            jax.ShapeDtypeStruct((NROWS * K,), jnp.float32),
        ],
        mesh=mesh,
        compiler_params=pltpu.CompilerParams(needs_layout_passes=False),
        scratch_types=[
            pltpu.VMEM((CHUNK * K,), jnp.int32),
            pltpu.VMEM((CHUNK, EMB_DIM), jnp.float32),
            pltpu.VMEM((CHUNK, EMB_DIM), jnp.float32),
            pltpu.VMEM((CHUNK * K, EMB_DIM), jnp.float32),
            pltpu.VMEM((CHUNK * K,), jnp.float32),
            pltpu.VMEM((CHUNK * K,), jnp.float32),
            pltpu.SemaphoreType.DMA,
        ],
    )
    def k(e_hbm, q_hbm, f_hbm, samp_hbm, d2_out, lg_out,
          idx_v, q_v, f_v, s_v, d2_v, lg_v, sem):
        wid = lax.axis_index("s") * 2 + lax.axis_index("c")
        h = wid // WPH
        off = h * N_NODES
        lane_iota = lax.iota(jnp.int32, LANES)

        def chunk_body(c, carry):
            qbase = wid * RPW + c * CHUNK
            pltpu.sync_copy(q_hbm.at[pl.ds(qbase, CHUNK)], q_v)
            pltpu.sync_copy(f_hbm.at[pl.ds(qbase, CHUNK)], f_v)
            pltpu.sync_copy(samp_hbm.at[pl.ds(qbase * K, CHUNK * K)], idx_v)
            for i in range((CHUNK * K) // LANES):
                sl = pl.ds(i * LANES, LANES)
                idx_v[sl] = idx_v[sl] + off
            pltpu.async_copy(e_hbm.at[idx_v], s_v, sem).wait()

            def pair_body(p, carry):
                resd2 = jnp.zeros((LANES,), jnp.float32)
                reslg = jnp.zeros((LANES,), jnp.float32)
                for qq in range(2):
                    i = 2 * p + qq
                    qregs = [q_v[i, pl.ds(j * LANES, LANES)]
                             for j in range(EMB_DIM // LANES)]
                    fregs = [f_v[i, pl.ds(j * LANES, LANES)]
                             for j in range(EMB_DIM // LANES)]
                    for kk in range(K):
                        t = i * K + kk
                        d2acc = jnp.zeros((LANES,), jnp.float32)
                        lgacc = jnp.zeros((LANES,), jnp.float32)
                        for j in range(EMB_DIM // LANES):
                            sv = s_v[t, pl.ds(j * LANES, LANES)]
                            dv = qregs[j] - sv
                            d2acc = d2acc + dv * dv
                            lgacc = lgacc + dv * fregs[j]
                        hit = lane_iota == (qq * K + kk)
                        resd2 = jnp.where(hit, jnp.sum(d2acc), resd2)
                        reslg = jnp.where(hit, jnp.sum(lgacc), reslg)
                gsl = pl.ds(p * LANES, LANES)
                d2_v[gsl] = resd2
                lg_v[gsl] = reslg
                return carry

            lax.fori_loop(0, (CHUNK * K) // LANES, pair_body, 0)
            pltpu.sync_copy(d2_v, d2_out.at[pl.ds(qbase * K, CHUNK * K)])
            pltpu.sync_copy(lg_v, lg_out.at[pl.ds(qbase * K, CHUNK * K)])
            return carry

        lax.fori_loop(0, NCH, chunk_body, 0)

    return k(embeds_flat, qrows, frows, samp_flat)


def _combine_body(d2_ref, lg_ref, out_ref):
    d2 = d2_ref[...]
    lg = lg_ref[...]
    dist = jnp.sqrt(d2)
    e = jnp.exp(1.0 - dist)
    num = jnp.sum(e * lg, axis=2)
    den = jnp.sum(e, axis=2)
    num_t = num[:, :N_BATCH] + num[:, N_BATCH:]
    den_t = den[:, :N_BATCH] + den[:, N_BATCH:] + float(N_SENTINELS)
    softmin = num_t / den_t
    preds = jnp.mean(softmin, axis=0)
    out_ref[...] = (1.0 / (1.0 + jnp.exp(-preds)))[None, :]


def _combine(d2, lg):
    return pl.pallas_call(
        _combine_body,
        out_shape=jax.ShapeDtypeStruct((1, N_BATCH), jnp.float32),
    )(d2, lg)


@jax.jit
def kernel(adj_t, edges, embeds, field):
    nodes = jnp.concatenate([edges[0], edges[1]]).astype(jnp.int32)
    embeds_flat = embeds.reshape(N_HEADS * N_NODES, EMB_DIM)
    field_flat = field.reshape(N_HEADS * N_NODES, EMB_DIM)

    qrows, frows = _sc_gather_qf(embeds_flat, field_flat, nodes)

    nodes3 = nodes.reshape(NQ // QB, 1, QB)
    samples = _topk(qrows.reshape(N_HEADS, NQ, EMB_DIM), embeds, nodes3)
    samp_flat = samples.reshape(NROWS * N_NEAREST)

    d2, lg = _sc_dots(embeds_flat, qrows, frows, samp_flat)
    out = _combine(d2.reshape(N_HEADS, NQ, N_NEAREST),
                   lg.reshape(N_HEADS, NQ, N_NEAREST))
    return out.reshape(N_BATCH)
